# baseline (speedup 1.0000x reference)
import functools

import jax
import jax.numpy as jnp
from jax import lax
from jax.experimental import pallas as pl
from jax.experimental.pallas import tpu as pltpu
from jax.experimental.pallas import tpu_sc as plsc

N = 10000
E = 160000
F = 16
FA = 2 * F

NC, NS = 2, 16
NW = NC * NS
EPW = E // NW
SCHUNK = 1000
RPT = N // NS

PK = 8
EP = E // PK
BP = 1000
ZW = PK * F * F
OW = PK * FA

_sc_mesh = plsc.VectorSubcoreMesh(
    core_axis_name="c", subcore_axis_name="s", num_cores=NC, num_subcores=NS)


def _gather_body(v_hbm, ei_hbm, e_hbm, xj_hbm, el_hbm, idx_v, rows_v,
                 sem, esem):
    c = lax.axis_index("c")
    s = lax.axis_index("s")
    wid = s * NC + c
    base = pl.multiple_of(wid * EPW, 8)
    ecp = pltpu.async_copy(e_hbm.at[pl.ds(base, EPW)],
                           el_hbm.at[pl.ds(base, EPW)], esem)
    pltpu.sync_copy(ei_hbm.at[0, pl.ds(base, EPW)], idx_v)
    pltpu.async_copy(v_hbm.at[idx_v], rows_v, sem).wait()
    pltpu.sync_copy(rows_v, xj_hbm.at[pl.ds(base, EPW)])
    ecp.wait()


_gather = pl.kernel(
    _gather_body,
    out_type=(jax.ShapeDtypeStruct((E, F), jnp.float32),
              jax.ShapeDtypeStruct((E, F), jnp.float32)),
    mesh=_sc_mesh,
    compiler_params=pltpu.CompilerParams(use_tc_tiling_on_sc=False),
    scratch_types=[
        pltpu.VMEM((EPW,), jnp.int32),
        pltpu.VMEM((EPW, F), jnp.float32),
        pltpu.SemaphoreType.DMA,
        pltpu.SemaphoreType.DMA,
    ],
)


def _msg_body(xj_ref, e_ref, r8_ref, t8_ref, w8_ref, b8_ref, c8_ref, out_ref):
    xj8 = xj_ref[...].reshape(BP, PK * F)
    e8 = e_ref[...].reshape(BP, PK * F)
    xr = jnp.dot(xj8, r8_ref[...], preferred_element_type=jnp.float32)
    et = jnp.dot(e8, t8_ref[...], preferred_element_type=jnp.float32)
    z8 = xr * et
    m8 = (jnp.dot(z8, w8_ref[...], preferred_element_type=jnp.float32)
          + jnp.dot(xj8, b8_ref[...], preferred_element_type=jnp.float32))
    out_ref[...] = m8 + c8_ref[0:1, :]


def _messages(xj8, e8, r8, t8, w8, b8, c8):
    return pl.pallas_call(
        _msg_body,
        grid=(EP // BP,),
        in_specs=[
            pl.BlockSpec((BP * PK * F,), lambda i: (i,)),
            pl.BlockSpec((BP * PK * F,), lambda i: (i,)),
            pl.BlockSpec((PK * F, ZW), lambda i: (0, 0)),
            pl.BlockSpec((PK * F, ZW), lambda i: (0, 0)),
            pl.BlockSpec((ZW, OW), lambda i: (0, 0)),
            pl.BlockSpec((PK * F, OW), lambda i: (0, 0)),
            pl.BlockSpec((8, OW), lambda i: (0, 0)),
        ],
        out_specs=pl.BlockSpec((BP, OW), lambda i: (i, 0)),
        out_shape=jax.ShapeDtypeStruct((EP, OW), jnp.float32),
    )(xj8, e8, r8, t8, w8, b8, c8)


def _scatter_body(msg_hbm, ei_hbm, zero_hbm, part_hbm, acc_sh, idx_v, msg_v,
                  sem):
    c = lax.axis_index("c")
    s = lax.axis_index("s")
    wid = s * NC + c
    rbase = pl.multiple_of(s * RPT, 8)
    pltpu.sync_copy(zero_hbm, acc_sh.at[pl.ds(rbase, RPT)])
    plsc.subcore_barrier()

    def chunk(j, carry):
        base = pl.multiple_of(wid * EPW + j * SCHUNK, 8)
        pltpu.sync_copy(ei_hbm.at[1, pl.ds(base, SCHUNK)], idx_v)
        pltpu.sync_copy(msg_hbm.at[pl.ds(base, SCHUNK)], msg_v)
        pltpu.sync_copy(msg_v, acc_sh.at[idx_v], add=True)
        return carry

    lax.fori_loop(0, EPW // SCHUNK, chunk, 0)
    plsc.subcore_barrier()
    pltpu.sync_copy(acc_sh.at[pl.ds(rbase, RPT)],
                    part_hbm.at[c, pl.ds(rbase, RPT)])


_scatter = pl.kernel(
    _scatter_body,
    out_type=jax.ShapeDtypeStruct((NC, N, FA), jnp.float32),
    mesh=_sc_mesh,
    compiler_params=pltpu.CompilerParams(use_tc_tiling_on_sc=False),
    scratch_types=[
        pltpu.VMEM_SHARED((N, FA), jnp.float32),
        pltpu.VMEM((SCHUNK,), jnp.int32),
        pltpu.VMEM((SCHUNK, FA), jnp.float32),
        pltpu.SemaphoreType.DMA,
    ],
)


def _final_body(part_ref, v_ref, root_ref, bias_ref, out_ref):
    p0 = part_ref[0]
    p1 = part_ref[1]
    ssum = p0[:, 0:F] + p1[:, 0:F]
    cnt = p0[:, F:F + 1] + p1[:, F:F + 1]
    mean = ssum / jnp.maximum(cnt, 1.0)
    rt = jnp.dot(v_ref[...], root_ref[...], preferred_element_type=jnp.float32)
    out_ref[...] = mean + rt + bias_ref[...]


def _final(part, v, root, bias2d):
    return pl.pallas_call(
        _final_body,
        out_shape=jax.ShapeDtypeStruct((N, F), jnp.float32),
    )(part, v, root, bias2d)


def kernel(v, e, edge_index, W_enet, b_enet, root, bias):
    ei = edge_index.astype(jnp.int32)
    wr = jnp.transpose(W_enet.reshape(F, F, F), (1, 0, 2)).reshape(F * F, F)
    bm = b_enet.reshape(F, F)
    eye8 = jnp.eye(PK, dtype=jnp.float32)
    eye16 = jnp.eye(F, dtype=jnp.float32)
    ones_row = jnp.ones((1, F), jnp.float32)
    rmat = jnp.kron(eye16, ones_row)
    tmat = jnp.kron(ones_row, eye16)
    wr32 = jnp.pad(wr, ((0, 0), (0, F)))
    bm32 = jnp.pad(bm, ((0, 0), (0, F)))
    r8 = jnp.kron(eye8, rmat)
    t8 = jnp.kron(eye8, tmat)
    w8 = jnp.kron(eye8, wr32)
    b8 = jnp.kron(eye8, bm32)
    crow = jnp.kron(jnp.ones((PK,), jnp.float32),
                    (jnp.arange(FA) == F).astype(jnp.float32))
    c8 = jnp.broadcast_to(crow, (8, OW))
    zero = jnp.zeros((RPT, FA), jnp.float32)

    xj, el = _gather(v, ei, e)
    msg8 = _messages(xj.reshape(E * F), el.reshape(E * F),
                     r8, t8, w8, b8, c8)
    part = _scatter(msg8.reshape(E, FA), ei, zero)
    return _final(part, v, root, bias.reshape(1, F))


# speedup vs baseline: 2.3649x; 2.3649x over previous
import functools

import jax
import jax.numpy as jnp
from jax import lax
from jax.experimental import pallas as pl
from jax.experimental.pallas import tpu as pltpu
from jax.experimental.pallas import tpu_sc as plsc

N = 10000
E = 160000
F = 16
FA = 2 * F

NC, NS = 2, 16
NW = NC * NS
EPW = E // NW
SCHUNK = 1000
RPT = N // NS

PK = 8
EP = E // PK
BP = 1000
ZW = PK * F * F
OW = PK * FA

_sc_mesh = plsc.VectorSubcoreMesh(
    core_axis_name="c", subcore_axis_name="s", num_cores=NC, num_subcores=NS)


ECH = 1000


def _gather_body(v_hbm, ei_hbm, e_hbm, xj_hbm, el_hbm, idx_v, rows_v, ebuf_v,
                 sem, esem):
    c = lax.axis_index("c")
    s = lax.axis_index("s")
    wid = s * NC + c
    base = pl.multiple_of(wid * EPW, 8)
    pltpu.sync_copy(ei_hbm.at[0, pl.ds(base, EPW)], idx_v)
    gcp = pltpu.async_copy(v_hbm.at[idx_v], rows_v, sem)

    def echunk(j, carry):
        cb = pl.multiple_of(base + j * ECH, 8)
        pltpu.sync_copy(e_hbm.at[pl.ds(cb, ECH)], ebuf_v)
        pltpu.sync_copy(ebuf_v, el_hbm.at[pl.ds(cb, ECH)])
        return carry

    lax.fori_loop(0, EPW // ECH, echunk, 0)
    gcp.wait()
    pltpu.sync_copy(rows_v, xj_hbm.at[pl.ds(base, EPW)])


_gather = pl.kernel(
    _gather_body,
    out_type=(jax.ShapeDtypeStruct((E, F), jnp.float32),
              jax.ShapeDtypeStruct((E, F), jnp.float32)),
    mesh=_sc_mesh,
    compiler_params=pltpu.CompilerParams(use_tc_tiling_on_sc=False),
    scratch_types=[
        pltpu.VMEM((EPW,), jnp.int32),
        pltpu.VMEM((EPW, F), jnp.float32),
        pltpu.VMEM((ECH, F), jnp.float32),
        pltpu.SemaphoreType.DMA,
        pltpu.SemaphoreType.DMA,
    ],
)


def _msg_body(xj_ref, e_ref, r8_ref, t8_ref, w8_ref, b8_ref, c8_ref, out_ref):
    xj8 = xj_ref[...].reshape(BP, PK * F)
    e8 = e_ref[...].reshape(BP, PK * F)
    xr = jnp.dot(xj8, r8_ref[...], preferred_element_type=jnp.float32)
    et = jnp.dot(e8, t8_ref[...], preferred_element_type=jnp.float32)
    z8 = xr * et
    m8 = (jnp.dot(z8, w8_ref[...], preferred_element_type=jnp.float32)
          + jnp.dot(xj8, b8_ref[...], preferred_element_type=jnp.float32))
    m8 = m8 + c8_ref[0:1, :]
    out_ref[...] = m8.reshape(BP * OW)


def _messages(xj8, e8, r8, t8, w8, b8, c8):
    return pl.pallas_call(
        _msg_body,
        grid=(EP // BP,),
        in_specs=[
            pl.BlockSpec((BP * PK * F,), lambda i: (i,)),
            pl.BlockSpec((BP * PK * F,), lambda i: (i,)),
            pl.BlockSpec((PK * F, ZW), lambda i: (0, 0)),
            pl.BlockSpec((PK * F, ZW), lambda i: (0, 0)),
            pl.BlockSpec((ZW, OW), lambda i: (0, 0)),
            pl.BlockSpec((PK * F, OW), lambda i: (0, 0)),
            pl.BlockSpec((8, OW), lambda i: (0, 0)),
        ],
        out_specs=pl.BlockSpec((BP * OW,), lambda i: (i,)),
        out_shape=jax.ShapeDtypeStruct((EP * OW,), jnp.float32),
    )(xj8, e8, r8, t8, w8, b8, c8)


def _scatter_body(msg_hbm, ei_hbm, zero_hbm, part_hbm, acc_sh, idx_v, msg_v,
                  sem):
    c = lax.axis_index("c")
    s = lax.axis_index("s")
    wid = s * NC + c
    rbase = pl.multiple_of(s * RPT, 8)
    pltpu.sync_copy(zero_hbm, acc_sh.at[pl.ds(rbase, RPT)])
    plsc.subcore_barrier()

    def chunk(j, carry):
        base = pl.multiple_of(wid * EPW + j * SCHUNK, 8)
        pltpu.sync_copy(ei_hbm.at[1, pl.ds(base, SCHUNK)], idx_v)
        pltpu.sync_copy(msg_hbm.at[pl.ds(base, SCHUNK)], msg_v)
        pltpu.sync_copy(msg_v, acc_sh.at[idx_v], add=True)
        return carry

    lax.fori_loop(0, EPW // SCHUNK, chunk, 0)
    plsc.subcore_barrier()
    pltpu.sync_copy(acc_sh.at[pl.ds(rbase, RPT)],
                    part_hbm.at[c, pl.ds(rbase, RPT)])


_scatter = pl.kernel(
    _scatter_body,
    out_type=jax.ShapeDtypeStruct((NC, N, FA), jnp.float32),
    mesh=_sc_mesh,
    compiler_params=pltpu.CompilerParams(use_tc_tiling_on_sc=False),
    scratch_types=[
        pltpu.VMEM_SHARED((N, FA), jnp.float32),
        pltpu.VMEM((SCHUNK,), jnp.int32),
        pltpu.VMEM((SCHUNK, FA), jnp.float32),
        pltpu.SemaphoreType.DMA,
    ],
)


NPK = N // PK


def _final_body(part_ref, vl_ref, sel_s_ref, sel_c_ref, r8root_ref, b_ref,
                out_ref):
    p = part_ref[...].reshape(2 * NPK, OW)
    psum = p[0:NPK, :] + p[NPK:2 * NPK, :]
    s_pk = jnp.dot(psum, sel_s_ref[...], preferred_element_type=jnp.float32)
    cnt_pk = jnp.dot(psum, sel_c_ref[...], preferred_element_type=jnp.float32)
    mean = s_pk / jnp.maximum(cnt_pk, 1.0)
    v_pk = vl_ref[...].reshape(NPK, PK * F)
    rt = jnp.dot(v_pk, r8root_ref[...], preferred_element_type=jnp.float32)
    out_ref[...] = mean + rt + b_ref[0:1, :]


def _final(part_flat, vl_flat, sel_s, sel_c, r8root, brow):
    return pl.pallas_call(
        _final_body,
        out_shape=jax.ShapeDtypeStruct((NPK, PK * F), jnp.float32),
    )(part_flat, vl_flat, sel_s, sel_c, r8root, brow)


def kernel(v, e, edge_index, W_enet, b_enet, root, bias):
    ei = edge_index.astype(jnp.int32)
    wr = jnp.transpose(W_enet.reshape(F, F, F), (1, 0, 2)).reshape(F * F, F)
    bm = b_enet.reshape(F, F)
    eye8 = jnp.eye(PK, dtype=jnp.float32)
    eye16 = jnp.eye(F, dtype=jnp.float32)
    ones_row = jnp.ones((1, F), jnp.float32)
    rmat = jnp.kron(eye16, ones_row)
    tmat = jnp.kron(ones_row, eye16)
    wr32 = jnp.pad(wr, ((0, 0), (0, F)))
    bm32 = jnp.pad(bm, ((0, 0), (0, F)))
    r8 = jnp.kron(eye8, rmat)
    t8 = jnp.kron(eye8, tmat)
    w8 = jnp.kron(eye8, wr32)
    b8 = jnp.kron(eye8, bm32)
    crow = jnp.kron(jnp.ones((PK,), jnp.float32),
                    (jnp.arange(FA) == F).astype(jnp.float32))
    c8 = jnp.broadcast_to(crow, (8, OW))
    zero = jnp.zeros((RPT, FA), jnp.float32)
    sel1 = jnp.concatenate([eye16, jnp.zeros((F, F), jnp.float32)], axis=0)
    selc1 = (jnp.arange(FA)[:, None] == F).astype(jnp.float32) * jnp.ones(
        (1, F), jnp.float32)
    sel_s = jnp.kron(eye8, sel1)
    sel_c = jnp.kron(eye8, selc1)
    r8root = jnp.kron(eye8, root)
    brow = jnp.broadcast_to(jnp.kron(jnp.ones((PK,), jnp.float32), bias),
                            (8, PK * F))

    xj, el = _gather(v, ei, e)
    msg8 = _messages(xj.reshape(E * F), el.reshape(E * F),
                     r8, t8, w8, b8, c8)
    part = _scatter(msg8.reshape(E, FA), ei, zero)
    out_pk = _final(part.reshape(NC * N * FA), v.reshape(N * F),
                    sel_s, sel_c, r8root, brow)
    return out_pk.reshape(N, F)


# speedup vs baseline: 2.5050x; 1.0592x over previous
import functools

import jax
import jax.numpy as jnp
from jax import lax
from jax.experimental import pallas as pl
from jax.experimental.pallas import tpu as pltpu
from jax.experimental.pallas import tpu_sc as plsc

N = 10000
E = 160000
F = 16
FA = 2 * F

NC, NS = 2, 16
NW = NC * NS
EPW = E // NW
SCHUNK = 1000
RPT = N // NS

PK = 8
EP = E // PK
BP = 1000
ZW = PK * F * F
OW = PK * FA

_sc_mesh = plsc.VectorSubcoreMesh(
    core_axis_name="c", subcore_axis_name="s", num_cores=NC, num_subcores=NS)


def _gather_body(v_hbm, ei_hbm, xj_hbm, idx_v, rows_v, sem):
    c = lax.axis_index("c")
    s = lax.axis_index("s")
    wid = s * NC + c
    base = pl.multiple_of(wid * EPW, 8)
    pltpu.sync_copy(ei_hbm.at[0, pl.ds(base, EPW)], idx_v)
    pltpu.async_copy(v_hbm.at[idx_v], rows_v, sem).wait()
    pltpu.sync_copy(rows_v, xj_hbm.at[pl.ds(base, EPW)])


_gather = pl.kernel(
    _gather_body,
    out_type=jax.ShapeDtypeStruct((E, F), jnp.float32),
    mesh=_sc_mesh,
    compiler_params=pltpu.CompilerParams(use_tc_tiling_on_sc=False),
    scratch_types=[
        pltpu.VMEM((EPW,), jnp.int32),
        pltpu.VMEM((EPW, F), jnp.float32),
        pltpu.SemaphoreType.DMA,
    ],
)


def _msg_body(xj_ref, e_ref, r8_ref, t8_ref, w8_ref, b8_ref, c8_ref, out_ref):
    xj8 = xj_ref[...].reshape(BP, PK * F)
    e8 = e_ref[...].reshape(BP, PK * F)
    xr = jnp.dot(xj8, r8_ref[...], preferred_element_type=jnp.float32)
    et = jnp.dot(e8, t8_ref[...], preferred_element_type=jnp.float32)
    z8 = xr * et
    m8 = (jnp.dot(z8, w8_ref[...], preferred_element_type=jnp.float32)
          + jnp.dot(xj8, b8_ref[...], preferred_element_type=jnp.float32))
    m8 = m8 + c8_ref[0:1, :]
    out_ref[...] = m8.reshape(BP * OW)


def _messages(xj8, e8, r8, t8, w8, b8, c8):
    return pl.pallas_call(
        _msg_body,
        grid=(EP // BP,),
        in_specs=[
            pl.BlockSpec((BP * PK * F,), lambda i: (i,)),
            pl.BlockSpec((BP * PK * F,), lambda i: (i,)),
            pl.BlockSpec((PK * F, ZW), lambda i: (0, 0)),
            pl.BlockSpec((PK * F, ZW), lambda i: (0, 0)),
            pl.BlockSpec((ZW, OW), lambda i: (0, 0)),
            pl.BlockSpec((PK * F, OW), lambda i: (0, 0)),
            pl.BlockSpec((8, OW), lambda i: (0, 0)),
        ],
        out_specs=pl.BlockSpec((BP * OW,), lambda i: (i,)),
        out_shape=jax.ShapeDtypeStruct((EP * OW,), jnp.float32),
    )(xj8, e8, r8, t8, w8, b8, c8)


def _scatter_body(msg_hbm, ei_hbm, zero_hbm, part_hbm, acc_sh, idx_v, msg_v,
                  sem):
    c = lax.axis_index("c")
    s = lax.axis_index("s")
    wid = s * NC + c
    rbase = pl.multiple_of(s * RPT, 8)
    pltpu.sync_copy(zero_hbm, acc_sh.at[pl.ds(rbase, RPT)])
    plsc.subcore_barrier()

    def chunk(j, carry):
        base = pl.multiple_of(wid * EPW + j * SCHUNK, 8)
        pltpu.sync_copy(ei_hbm.at[1, pl.ds(base, SCHUNK)], idx_v)
        pltpu.sync_copy(msg_hbm.at[pl.ds(base, SCHUNK)], msg_v)
        pltpu.sync_copy(msg_v, acc_sh.at[idx_v], add=True)
        return carry

    lax.fori_loop(0, EPW // SCHUNK, chunk, 0)
    plsc.subcore_barrier()
    pltpu.sync_copy(acc_sh.at[pl.ds(rbase, RPT)],
                    part_hbm.at[c, pl.ds(rbase, RPT)])


_scatter = pl.kernel(
    _scatter_body,
    out_type=jax.ShapeDtypeStruct((NC, N, FA), jnp.float32),
    mesh=_sc_mesh,
    compiler_params=pltpu.CompilerParams(use_tc_tiling_on_sc=False),
    scratch_types=[
        pltpu.VMEM_SHARED((N, FA), jnp.float32),
        pltpu.VMEM((SCHUNK,), jnp.int32),
        pltpu.VMEM((SCHUNK, FA), jnp.float32),
        pltpu.SemaphoreType.DMA,
    ],
)


NPK = N // PK


def _final_body(part_ref, vl_ref, sel_s_ref, sel_c_ref, r8root_ref, b_ref,
                out_ref):
    p = part_ref[...].reshape(2 * NPK, OW)
    psum = p[0:NPK, :] + p[NPK:2 * NPK, :]
    s_pk = jnp.dot(psum, sel_s_ref[...], preferred_element_type=jnp.float32)
    cnt_pk = jnp.dot(psum, sel_c_ref[...], preferred_element_type=jnp.float32)
    mean = s_pk / jnp.maximum(cnt_pk, 1.0)
    v_pk = vl_ref[...].reshape(NPK, PK * F)
    rt = jnp.dot(v_pk, r8root_ref[...], preferred_element_type=jnp.float32)
    out_ref[...] = mean + rt + b_ref[0:1, :]


def _final(part_flat, vl_flat, sel_s, sel_c, r8root, brow):
    return pl.pallas_call(
        _final_body,
        out_shape=jax.ShapeDtypeStruct((NPK, PK * F), jnp.float32),
    )(part_flat, vl_flat, sel_s, sel_c, r8root, brow)


def kernel(v, e, edge_index, W_enet, b_enet, root, bias):
    ei = edge_index.astype(jnp.int32)
    wr = jnp.transpose(W_enet.reshape(F, F, F), (1, 0, 2)).reshape(F * F, F)
    bm = b_enet.reshape(F, F)
    eye8 = jnp.eye(PK, dtype=jnp.float32)
    eye16 = jnp.eye(F, dtype=jnp.float32)
    ones_row = jnp.ones((1, F), jnp.float32)
    rmat = jnp.kron(eye16, ones_row)
    tmat = jnp.kron(ones_row, eye16)
    wr32 = jnp.pad(wr, ((0, 0), (0, F)))
    bm32 = jnp.pad(bm, ((0, 0), (0, F)))
    r8 = jnp.kron(eye8, rmat)
    t8 = jnp.kron(eye8, tmat)
    w8 = jnp.kron(eye8, wr32)
    b8 = jnp.kron(eye8, bm32)
    crow = jnp.kron(jnp.ones((PK,), jnp.float32),
                    (jnp.arange(FA) == F).astype(jnp.float32))
    c8 = jnp.broadcast_to(crow, (8, OW))
    zero = jnp.zeros((RPT, FA), jnp.float32)
    sel1 = jnp.concatenate([eye16, jnp.zeros((F, F), jnp.float32)], axis=0)
    selc1 = (jnp.arange(FA)[:, None] == F).astype(jnp.float32) * jnp.ones(
        (1, F), jnp.float32)
    sel_s = jnp.kron(eye8, sel1)
    sel_c = jnp.kron(eye8, selc1)
    r8root = jnp.kron(eye8, root)
    brow = jnp.broadcast_to(jnp.kron(jnp.ones((PK,), jnp.float32), bias),
                            (8, PK * F))

    xj = _gather(v, ei)
    msg8 = _messages(xj.reshape(E * F), e.reshape(E * F),
                     r8, t8, w8, b8, c8)
    part = _scatter(msg8.reshape(E, FA), ei, zero)
    out_pk = _final(part.reshape(NC * N * FA), v.reshape(N * F),
                    sel_s, sel_c, r8root, brow)
    return out_pk.reshape(N, F)
